# Initial kernel scaffold; baseline (speedup 1.0000x reference)
import functools

import jax
import jax.numpy as jnp
from jax import lax
from jax.experimental import pallas as pl
from jax.experimental.pallas import tpu as pltpu
from jax.experimental.pallas import tpu_sc as plsc

N_NODES = 10000
N_EDGES = 320000
EMB = 128

NC = 2
NS = 16
NW = NC * NS
EPW = N_EDGES // NW
C = 80
CHUNKS = EPW // C
ZROWS = 125
RPT = N_NODES // NS
RCP = 5


@functools.partial(
    pl.kernel,
    mesh=plsc.VectorSubcoreMesh(core_axis_name="c", subcore_axis_name="s"),
    out_type=jax.ShapeDtypeStruct((NC, N_NODES, EMB), jnp.float32),
    scratch_types=[
        pltpu.VMEM((CHUNKS, C), jnp.int32),
        pltpu.VMEM((CHUNKS, C), jnp.int32),
        pltpu.VMEM((C, EMB), jnp.float32),
        pltpu.VMEM((C, EMB), jnp.float32),
        pltpu.VMEM((ZROWS, EMB), jnp.float32),
        pltpu.VMEM_SHARED((N_NODES, EMB), jnp.float32),
        pltpu.SemaphoreType.DMA,
    ],
)
def _edge_agg(x_hbm, src_hbm, dst_hbm, ea_hbm, out_hbm,
              src_v, dst_v, rows_v, ea_v, zbuf, agg_sh, sem):
    c = lax.axis_index("c")
    s = lax.axis_index("s")
    w = c * NS + s

    def _zrow(i, _):
        def _zcol(k, _):
            zbuf[i, pl.ds(k * 16, 16)] = jnp.zeros((16,), jnp.float32)
            return 0
        return lax.fori_loop(0, EMB // 16, _zcol, 0)
    lax.fori_loop(0, ZROWS, _zrow, 0)

    def _zchunk(j, _):
        pltpu.sync_copy(zbuf, agg_sh.at[pl.ds(s * RPT + j * ZROWS, ZROWS)])
        return 0
    lax.fori_loop(0, RCP, _zchunk, 0)
    plsc.subcore_barrier()

    pltpu.sync_copy(src_hbm.at[w], src_v)
    pltpu.sync_copy(dst_hbm.at[w], dst_v)

    def _chunk(j, _):
        cp = pltpu.async_copy(x_hbm.at[src_v.at[j]], rows_v, sem)
        pltpu.sync_copy(ea_hbm.at[w * CHUNKS + j], ea_v)
        cp.wait()

        def _row(r, _):
            def _col(k, _):
                v = rows_v[r, pl.ds(k * 16, 16)] + ea_v[r, pl.ds(k * 16, 16)]
                rows_v[r, pl.ds(k * 16, 16)] = jnp.maximum(v, 0.0)
                return 0
            return lax.fori_loop(0, EMB // 16, _col, 0)
        lax.fori_loop(0, C, _row, 0)

        pltpu.sync_copy(rows_v, agg_sh.at[dst_v.at[j]], add=True)
        return 0
    lax.fori_loop(0, CHUNKS, _chunk, 0)

    plsc.subcore_barrier()

    def _out(j, _):
        base = s * RPT + j * ZROWS
        pltpu.sync_copy(agg_sh.at[pl.ds(base, ZROWS)], zbuf)
        pltpu.sync_copy(zbuf, out_hbm.at[c].at[pl.ds(base, ZROWS)])
        return 0
    lax.fori_loop(0, RCP, _out, 0)


def _mlp_body(x_ref, a0_ref, a1_ref, w1_ref, b1_ref, w2_ref, b2_ref, o_ref):
    h = x_ref[...] + a0_ref[...] + a1_ref[...]
    h = jnp.dot(h, w1_ref[...], preferred_element_type=jnp.float32)
    h = jnp.maximum(h + b1_ref[...], 0.0)
    o_ref[...] = (
        jnp.dot(h, w2_ref[...], preferred_element_type=jnp.float32)
        + b2_ref[...]
    )


_ROW_BLK = 1000


def _mlp(x, a0, a1, W1, b1, W2, b2):
    return pl.pallas_call(
        _mlp_body,
        grid=(N_NODES // _ROW_BLK,),
        in_specs=[
            pl.BlockSpec((_ROW_BLK, EMB), lambda i: (i, 0)),
            pl.BlockSpec((_ROW_BLK, EMB), lambda i: (i, 0)),
            pl.BlockSpec((_ROW_BLK, EMB), lambda i: (i, 0)),
            pl.BlockSpec((EMB, 2 * EMB), lambda i: (0, 0)),
            pl.BlockSpec((1, 2 * EMB), lambda i: (0, 0)),
            pl.BlockSpec((2 * EMB, EMB), lambda i: (0, 0)),
            pl.BlockSpec((1, EMB), lambda i: (0, 0)),
        ],
        out_specs=pl.BlockSpec((_ROW_BLK, EMB), lambda i: (i, 0)),
        out_shape=jax.ShapeDtypeStruct((N_NODES, EMB), jnp.float32),
    )(x, a0, a1, W1, b1.reshape(1, -1), W2, b2.reshape(1, -1))


@jax.jit
def kernel(x, edge_index, edge_attr, W1, b1, W2, b2):
    src = edge_index[0].astype(jnp.int32).reshape(NW, CHUNKS, C)
    dst = edge_index[1].astype(jnp.int32).reshape(NW, CHUNKS, C)
    ea = edge_attr.reshape(NW * CHUNKS, C, EMB)
    partials = _edge_agg(x, src, dst, ea)
    return _mlp(x, partials[0], partials[1], W1, b1, W2, b2)


# speedup vs baseline: 4.1965x; 4.1965x over previous
import functools

import jax
import jax.numpy as jnp
from jax import lax
from jax.experimental import pallas as pl
from jax.experimental.pallas import tpu as pltpu
from jax.experimental.pallas import tpu_sc as plsc

N_NODES = 10000
N_EDGES = 320000
EMB = 128

NC = 2
NS = 16
NW = NC * NS
EPW = N_EDGES // NW
C = 80
CHUNKS = EPW // C
ZROWS = 80
NODE_CHUNKS = N_NODES // ZROWS
RR = -(-NODE_CHUNKS // NS)


@functools.partial(
    pl.kernel,
    mesh=plsc.VectorSubcoreMesh(core_axis_name="c", subcore_axis_name="s"),
    out_type=jax.ShapeDtypeStruct((NC, N_NODES, EMB), jnp.float32),
    scratch_types=[
        pltpu.VMEM((C,), jnp.int32),
        pltpu.VMEM((C,), jnp.int32),
        pltpu.VMEM((C, EMB), jnp.float32),
        pltpu.VMEM((C, EMB), jnp.float32),
        pltpu.VMEM((ZROWS, EMB), jnp.float32),
        pltpu.VMEM_SHARED((N_NODES, EMB), jnp.float32),
        pltpu.SemaphoreType.DMA,
    ],
)
def _edge_agg(x_hbm, src_hbm, dst_hbm, ea_hbm, out_hbm,
              src_v, dst_v, rows_v, ea_v, zbuf, agg_sh, sem):
    c = lax.axis_index("c")
    s = lax.axis_index("s")
    w = c * NS + s

    def _zrow(i, _):
        def _zcol(k, _):
            zbuf[i, pl.ds(k * 16, 16)] = jnp.zeros((16,), jnp.float32)
            return 0
        return lax.fori_loop(0, EMB // 16, _zcol, 0)
    lax.fori_loop(0, ZROWS, _zrow, 0)

    def _zchunk(i, _):
        j = s + i * NS

        @pl.when(j < NODE_CHUNKS)
        def _():
            pltpu.sync_copy(zbuf, agg_sh.at[pl.ds(j * ZROWS, ZROWS)])
        return 0
    lax.fori_loop(0, RR, _zchunk, 0)
    plsc.subcore_barrier()

    def _chunk(j, _):
        pltpu.sync_copy(src_hbm.at[w * CHUNKS + j], src_v)
        cp = pltpu.async_copy(x_hbm.at[src_v], rows_v, sem)
        pltpu.sync_copy(dst_hbm.at[w * CHUNKS + j], dst_v)
        pltpu.sync_copy(ea_hbm.at[w * CHUNKS + j], ea_v)
        cp.wait()

        def _row(r, _):
            def _col(k, _):
                v = rows_v[r, pl.ds(k * 16, 16)] + ea_v[r, pl.ds(k * 16, 16)]
                rows_v[r, pl.ds(k * 16, 16)] = jnp.maximum(v, 0.0)
                return 0
            return lax.fori_loop(0, EMB // 16, _col, 0)
        lax.fori_loop(0, C, _row, 0)

        pltpu.sync_copy(rows_v, agg_sh.at[dst_v], add=True)
        return 0
    lax.fori_loop(0, CHUNKS, _chunk, 0)

    plsc.subcore_barrier()

    def _out(i, _):
        j = s + i * NS

        @pl.when(j < NODE_CHUNKS)
        def _():
            base = j * ZROWS
            pltpu.sync_copy(agg_sh.at[pl.ds(base, ZROWS)], zbuf)
            pltpu.sync_copy(zbuf, out_hbm.at[c].at[pl.ds(base, ZROWS)])
        return 0
    lax.fori_loop(0, RR, _out, 0)


def _mlp_body(x_ref, a0_ref, a1_ref, w1_ref, b1_ref, w2_ref, b2_ref, o_ref):
    h = x_ref[...] + a0_ref[...] + a1_ref[...]
    h = jnp.dot(h, w1_ref[...], preferred_element_type=jnp.float32)
    h = jnp.maximum(h + b1_ref[...], 0.0)
    o_ref[...] = (
        jnp.dot(h, w2_ref[...], preferred_element_type=jnp.float32)
        + b2_ref[...]
    )


_ROW_BLK = 1000


def _mlp(x, a0, a1, W1, b1, W2, b2):
    return pl.pallas_call(
        _mlp_body,
        grid=(N_NODES // _ROW_BLK,),
        in_specs=[
            pl.BlockSpec((_ROW_BLK, EMB), lambda i: (i, 0)),
            pl.BlockSpec((_ROW_BLK, EMB), lambda i: (i, 0)),
            pl.BlockSpec((_ROW_BLK, EMB), lambda i: (i, 0)),
            pl.BlockSpec((EMB, 2 * EMB), lambda i: (0, 0)),
            pl.BlockSpec((1, 2 * EMB), lambda i: (0, 0)),
            pl.BlockSpec((2 * EMB, EMB), lambda i: (0, 0)),
            pl.BlockSpec((1, EMB), lambda i: (0, 0)),
        ],
        out_specs=pl.BlockSpec((_ROW_BLK, EMB), lambda i: (i, 0)),
        out_shape=jax.ShapeDtypeStruct((N_NODES, EMB), jnp.float32),
    )(x, a0, a1, W1, b1.reshape(1, -1), W2, b2.reshape(1, -1))


@jax.jit
def kernel(x, edge_index, edge_attr, W1, b1, W2, b2):
    src = edge_index[0].astype(jnp.int32).reshape(NW * CHUNKS, C)
    dst = edge_index[1].astype(jnp.int32).reshape(NW * CHUNKS, C)
    ea = edge_attr.reshape(NW * CHUNKS, C, EMB)
    partials = _edge_agg(x, src, dst, ea)
    return _mlp(x, partials[0], partials[1], W1, b1, W2, b2)


# speedup vs baseline: 4.3368x; 1.0334x over previous
import functools

import jax
import jax.numpy as jnp
from jax import lax
from jax.experimental import pallas as pl
from jax.experimental.pallas import tpu as pltpu
from jax.experimental.pallas import tpu_sc as plsc

N_NODES = 10000
N_EDGES = 320000
EMB = 128

NC = 2
NS = 16
NW = NC * NS
EPW = N_EDGES // NW
C = 80
CHUNKS = EPW // C
ZROWS = 80
NODE_CHUNKS = N_NODES // ZROWS
RR = -(-NODE_CHUNKS // NS)


@functools.partial(
    pl.kernel,
    mesh=plsc.VectorSubcoreMesh(core_axis_name="c", subcore_axis_name="s"),
    out_type=jax.ShapeDtypeStruct((NC, N_NODES, EMB), jnp.float32),
    scratch_types=[
        pltpu.VMEM((2, C), jnp.int32),
        pltpu.VMEM((C, EMB), jnp.float32),
        pltpu.VMEM((C, EMB), jnp.float32),
        pltpu.VMEM((ZROWS, EMB), jnp.float32),
        pltpu.VMEM_SHARED((N_NODES, EMB), jnp.float32),
        pltpu.SemaphoreType.DMA,
    ],
)
def _edge_agg(x_hbm, idx_hbm, ea_hbm, out_hbm,
              idx_v, rows_v, ea_v, zbuf, agg_sh, sem):
    c = lax.axis_index("c")
    s = lax.axis_index("s")
    w = c * NS + s

    def _zrow(i, _):
        def _zcol(k, _):
            zbuf[i, pl.ds(k * 16, 16)] = jnp.zeros((16,), jnp.float32)
            return 0
        return lax.fori_loop(0, EMB // 16, _zcol, 0)
    lax.fori_loop(0, ZROWS, _zrow, 0)

    def _zchunk(i, _):
        j = s + i * NS

        @pl.when(j < NODE_CHUNKS)
        def _():
            pltpu.sync_copy(zbuf, agg_sh.at[pl.ds(j * ZROWS, ZROWS)])
        return 0
    lax.fori_loop(0, RR, _zchunk, 0)
    plsc.subcore_barrier()

    def _chunk(j, _):
        pltpu.sync_copy(idx_hbm.at[w * CHUNKS + j], idx_v)
        cp = pltpu.async_copy(x_hbm.at[idx_v.at[0]], rows_v, sem)
        pltpu.sync_copy(ea_hbm.at[w * CHUNKS + j], ea_v)
        cp.wait()

        def _row(r, _):
            for k in range(EMB // 16):
                v = rows_v[r, pl.ds(k * 16, 16)] + ea_v[r, pl.ds(k * 16, 16)]
                rows_v[r, pl.ds(k * 16, 16)] = jnp.maximum(v, 0.0)
            return 0
        lax.fori_loop(0, C, _row, 0)

        pltpu.sync_copy(rows_v, agg_sh.at[idx_v.at[1]], add=True)
        return 0
    lax.fori_loop(0, CHUNKS, _chunk, 0)

    plsc.subcore_barrier()

    def _out(i, _):
        j = s + i * NS

        @pl.when(j < NODE_CHUNKS)
        def _():
            base = j * ZROWS
            pltpu.sync_copy(agg_sh.at[pl.ds(base, ZROWS)], zbuf)
            pltpu.sync_copy(zbuf, out_hbm.at[c].at[pl.ds(base, ZROWS)])
        return 0
    lax.fori_loop(0, RR, _out, 0)


def _mlp_body(x_ref, a0_ref, a1_ref, w1_ref, b1_ref, w2_ref, b2_ref, o_ref):
    h = x_ref[...] + a0_ref[...] + a1_ref[...]
    h = jnp.dot(h, w1_ref[...], preferred_element_type=jnp.float32)
    h = jnp.maximum(h + b1_ref[...], 0.0)
    o_ref[...] = (
        jnp.dot(h, w2_ref[...], preferred_element_type=jnp.float32)
        + b2_ref[...]
    )


_ROW_BLK = 1000


def _mlp(x, a0, a1, W1, b1, W2, b2):
    return pl.pallas_call(
        _mlp_body,
        grid=(N_NODES // _ROW_BLK,),
        in_specs=[
            pl.BlockSpec((_ROW_BLK, EMB), lambda i: (i, 0)),
            pl.BlockSpec((_ROW_BLK, EMB), lambda i: (i, 0)),
            pl.BlockSpec((_ROW_BLK, EMB), lambda i: (i, 0)),
            pl.BlockSpec((EMB, 2 * EMB), lambda i: (0, 0)),
            pl.BlockSpec((1, 2 * EMB), lambda i: (0, 0)),
            pl.BlockSpec((2 * EMB, EMB), lambda i: (0, 0)),
            pl.BlockSpec((1, EMB), lambda i: (0, 0)),
        ],
        out_specs=pl.BlockSpec((_ROW_BLK, EMB), lambda i: (i, 0)),
        out_shape=jax.ShapeDtypeStruct((N_NODES, EMB), jnp.float32),
    )(x, a0, a1, W1, b1.reshape(1, -1), W2, b2.reshape(1, -1))


@jax.jit
def kernel(x, edge_index, edge_attr, W1, b1, W2, b2):
    ei = edge_index.astype(jnp.int32).reshape(2, NW * CHUNKS, C)
    idx = jnp.swapaxes(ei, 0, 1)
    ea = edge_attr.reshape(NW * CHUNKS, C, EMB)
    partials = _edge_agg(x, idx, ea)
    return _mlp(x, partials[0], partials[1], W1, b1, W2, b2)
